# baseline (speedup 1.0000x reference)
import functools

import jax
import jax.numpy as jnp
from jax import lax
from jax.experimental import pallas as pl
from jax.experimental.pallas import tpu as pltpu
from jax.experimental.pallas import tpu_sc as plsc

_NC = 2
_NS = 16
_L = 16
_EMB = 64
_K = 128

_GATHER_DN = lax.GatherDimensionNumbers(
    offset_dims=(), collapsed_slice_dims=(0,), start_index_map=(0,))


@functools.cache
def _make_layer(n_total: int, n_half: int, nb: int):
    zrows_per_tile = (n_half // _NS + 15) // 8 * 8
    r_pad = zrows_per_tile * _NS
    orows_per_tile = n_half // _NS // 8 * 8
    otail = n_half - orows_per_tile * _NS
    ntrip = nb // 3
    mesh = plsc.VectorSubcoreMesh(
        core_axis_name="c", subcore_axis_name="s", num_cores=_NC
    )

    @functools.partial(
        pl.kernel,
        out_type=jax.ShapeDtypeStruct((n_total, _EMB), jnp.float32),
        mesh=mesh,
        compiler_params=pltpu.CompilerParams(
            use_tc_tiling_on_sc=False, needs_layout_passes=False),
        scratch_types=[
            pltpu.VMEM((3, _K), jnp.int32),
            pltpu.VMEM((3, _K), jnp.int32),
            pltpu.VMEM((3, _K), jnp.int32),
            pltpu.VMEM((_K, _EMB), jnp.float32),
            pltpu.VMEM((_K, _EMB), jnp.float32),
            pltpu.VMEM((_K, _EMB), jnp.float32),
            pltpu.VMEM_SHARED((r_pad, _EMB), jnp.float32),
            pltpu.SemaphoreType.DMA,
            pltpu.SemaphoreType.DMA,
            pltpu.SemaphoreType.DMA,
            pltpu.SemaphoreType.DMA,
            pltpu.SemaphoreType.DMA,
            pltpu.SemaphoreType.DMA,
        ],
    )
    def layer(x_hbm, meta_hbm, y_hbm,
              meta0, meta1, meta2, row0, row1, row2, acc,
              isem0, isem1, isem2, gsem0, gsem1, gsem2):
        core = lax.axis_index("c")
        sub = lax.axis_index("s")
        base = core * n_half
        metas = (meta0, meta1, meta2)
        rows = (row0, row1, row2)
        isems = (isem0, isem1, isem2)
        gsems = (gsem0, gsem1, gsem2)

        zero_v = row0
        zv = jnp.zeros((_L,), jnp.float32)

        def zfill(i, c):
            for g in range(_EMB // _L):
                zero_v[i, pl.ds(g * _L, _L)] = zv
            return c

        lax.fori_loop(0, 128, zfill, 0)
        z0 = sub * zrows_per_tile
        nfull = zrows_per_tile // 128
        zrem = zrows_per_tile - nfull * 128

        def zcopy(i, c):
            pltpu.sync_copy(zero_v, acc.at[pl.ds(z0 + i * 128, 128)])
            return c

        lax.fori_loop(0, nfull, zcopy, 0)
        if zrem:
            pltpu.sync_copy(zero_v.at[pl.ds(0, zrem)],
                            acc.at[pl.ds(z0 + nfull * 128, zrem)])
        plsc.subcore_barrier()

        def process(s):
            meta, row = metas[s], rows[s]
            for g in range(_K // _L):
                r16 = meta[1, pl.ds(g * _L, _L)]
                loc = r16 - base
                ok = (loc >= 0) & (loc < n_half)
                meta[1, pl.ds(g * _L, _L)] = jnp.where(
                    ok, loc, n_half + (r16 & 7))

            def scale(j16, c2):
                v16 = plsc.bitcast(meta[2, pl.ds(j16 * _L, _L)], jnp.float32)
                for jj in range(_L):
                    j = j16 * _L + jj
                    bj = lax.gather(
                        v16, jnp.full((_L, 1), jj, jnp.int32),
                        _GATHER_DN, slice_sizes=(1,),
                        mode=lax.GatherScatterMode.PROMISE_IN_BOUNDS)
                    for g in range(_EMB // _L):
                        row[j, pl.ds(g * _L, _L)] = (
                            row[j, pl.ds(g * _L, _L)] * bj
                        )
                return c2

            for j16 in range(_K // _L):
                scale(j16, 0)
            pltpu.sync_copy(row, acc.at[meta.at[1]], add=True)

        def meta_copy(b, s):
            return pltpu.make_async_copy(
                meta_hbm.at[sub, b], metas[s], isems[s])

        def gather_copy(s):
            return pltpu.make_async_copy(
                x_hbm.at[metas[s].at[0]], rows[s], gsems[s])

        pltpu.sync_copy(meta_hbm.at[sub, 0], meta0)
        pltpu.sync_copy(meta_hbm.at[sub, 1], meta1)
        gather_copy(0).start()
        gather_copy(1).start()
        meta_copy(2, 2).start()

        def step(i, b, s, guard):
            s2 = (s + 2) % 3
            if guard is None:
                meta_copy(b + 2, s2).wait()
                gather_copy(s2).start()
            else:
                @pl.when(guard)
                def _():
                    meta_copy(b + 2, s2).wait()
                    gather_copy(s2).start()

            gather_copy(s).wait()
            process(s)

            @pl.when(i < ntrip - 1)
            def _():
                meta_copy(b + 3, s).start()

        def triple(i, c):
            b0 = 3 * i
            step(i, b0, 0, None)
            step(i, b0 + 1, 1, i < ntrip - 1)
            step(i, b0 + 2, 2, i < ntrip - 1)
            return c

        lax.fori_loop(0, ntrip, triple, 0)
        plsc.subcore_barrier()

        o0 = sub * orows_per_tile
        pltpu.sync_copy(acc.at[pl.ds(o0, orows_per_tile)],
                        y_hbm.at[pl.ds(base + o0, orows_per_tile)])
        if otail:
            @pl.when(sub == _NS - 1)
            def _():
                pltpu.sync_copy(
                    acc.at[pl.ds(_NS * orows_per_tile, otail)],
                    y_hbm.at[pl.ds(base + _NS * orows_per_tile, otail)],
                )

    return layer


def _final_body(x0u, x1u, x2u, x3u, x0i, x1i, x2i, x3i, mr, w, b,
                uf, itf, mu, pa, mk):
    ufv = (x0u[...] + x1u[...] + x2u[...] + x3u[...]) * 0.25
    itv = (x0i[...] + x1i[...] + x2i[...] + x3i[...]) * 0.25
    m = (mr[...] > 0.2).astype(jnp.float32)
    muv = ufv * m
    pav = jnp.dot(muv, w[...], preferred_element_type=jnp.float32) + b[...]
    uf[...] = ufv
    itf[...] = itv
    mu[...] = muv
    pa[...] = jnp.maximum(pav, 0.0)
    mk[...] = m


def kernel(user_emb, item_emb, edge_index, adj_values, attr_W, attr_b,
           mask_rand):
    nu, emb = user_emb.shape
    ni = item_emb.shape[0]
    nt = nu + ni
    e = edge_index.shape[1]

    x0 = jnp.concatenate([user_emb, item_emb], axis=0)

    nbl = -(-e // (_NS * _K))
    nb = -(-nbl // 3) * 3
    e_pad = _NS * nb * _K
    pe = e_pad - e
    src_p = jnp.concatenate([edge_index[1], jnp.zeros((pe,), jnp.int32)])
    dst_p = jnp.concatenate([edge_index[0], jnp.full((pe,), -1, jnp.int32)])
    val_p = jnp.concatenate([
        lax.bitcast_convert_type(adj_values, jnp.int32),
        jnp.zeros((pe,), jnp.int32)])
    meta = jnp.stack([src_p.reshape(_NS, nb, _K),
                      dst_p.reshape(_NS, nb, _K),
                      val_p.reshape(_NS, nb, _K)], axis=2)

    layer = _make_layer(nt, nu, nb)
    xs = [x0]
    x = x0
    for _ in range(3):
        x = layer(x, meta)
        xs.append(x)

    blk = 1000
    ngrid = nu // blk
    ublk = pl.BlockSpec((blk, emb), lambda i: (i, 0))
    iblk = pl.BlockSpec((blk, emb), lambda i: (i + ngrid, 0))
    full = pl.BlockSpec((emb, emb), lambda i: (0, 0))
    brow = pl.BlockSpec((1, emb), lambda i: (0, 0))
    o = jax.ShapeDtypeStruct((nu, emb), jnp.float32)

    uf, itf, mu, pa, mk = pl.pallas_call(
        _final_body,
        grid=(ngrid,),
        in_specs=[ublk, ublk, ublk, ublk, iblk, iblk, iblk, iblk,
                  ublk, full, brow],
        out_specs=[ublk, ublk, ublk, ublk, ublk],
        out_shape=[o, o, o, o, o],
    )(xs[0], xs[1], xs[2], xs[3], xs[0], xs[1], xs[2], xs[3],
      mask_rand, attr_W, attr_b.reshape(1, emb))

    return (uf, itf, mu, pa, mk)


# speedup vs baseline: 1.2288x; 1.2288x over previous
import functools

import jax
import jax.numpy as jnp
from jax import lax
from jax.experimental import pallas as pl
from jax.experimental.pallas import tpu as pltpu
from jax.experimental.pallas import tpu_sc as plsc

_NC = 2
_NS = 16
_L = 16
_EMB = 64
_K = 128

_GATHER_DN = lax.GatherDimensionNumbers(
    offset_dims=(), collapsed_slice_dims=(0,), start_index_map=(0,))


@functools.cache
def _make_layer(n_total: int, n_half: int, nb: int):
    zrows_per_tile = (n_half // _NS + 15) // 8 * 8
    r_pad = zrows_per_tile * _NS
    orows_per_tile = n_half // _NS // 8 * 8
    otail = n_half - orows_per_tile * _NS
    npair = nb // 2
    mesh = plsc.VectorSubcoreMesh(
        core_axis_name="c", subcore_axis_name="s", num_cores=_NC
    )

    @functools.partial(
        pl.kernel,
        out_type=jax.ShapeDtypeStruct((n_total, _EMB), jnp.float32),
        mesh=mesh,
        compiler_params=pltpu.CompilerParams(
            use_tc_tiling_on_sc=False, needs_layout_passes=False),
        scratch_types=[
            pltpu.VMEM((3, _K), jnp.int32),
            pltpu.VMEM((3, _K), jnp.int32),
            pltpu.VMEM((_K, _EMB), jnp.float32),
            pltpu.VMEM((_K, _EMB), jnp.float32),
            pltpu.VMEM_SHARED((r_pad, _EMB), jnp.float32),
            pltpu.SemaphoreType.DMA,
            pltpu.SemaphoreType.DMA,
            pltpu.SemaphoreType.DMA,
            pltpu.SemaphoreType.DMA,
        ],
    )
    def layer(x_hbm, meta_hbm, y_hbm,
              meta0, meta1, row0, row1, acc,
              isem0, isem1, gsem0, gsem1):
        core = lax.axis_index("c")
        sub = lax.axis_index("s")
        base = core * n_half
        metas = (meta0, meta1)
        rows = (row0, row1)
        isems = (isem0, isem1)
        gsems = (gsem0, gsem1)

        zero_v = row0
        zv = jnp.zeros((_L,), jnp.float32)

        def zfill(i, c):
            for g in range(_EMB // _L):
                zero_v[i, pl.ds(g * _L, _L)] = zv
            return c

        lax.fori_loop(0, 128, zfill, 0)
        z0 = sub * zrows_per_tile
        nfull = zrows_per_tile // 128
        zrem = zrows_per_tile - nfull * 128

        def zcopy(i, c):
            pltpu.sync_copy(zero_v, acc.at[pl.ds(z0 + i * 128, 128)])
            return c

        lax.fori_loop(0, nfull, zcopy, 0)
        if zrem:
            pltpu.sync_copy(zero_v.at[pl.ds(0, zrem)],
                            acc.at[pl.ds(z0 + nfull * 128, zrem)])
        plsc.subcore_barrier()

        def process(s):
            meta, row = metas[s], rows[s]
            for g in range(_K // _L):
                r16 = meta[1, pl.ds(g * _L, _L)]
                loc = r16 - base
                ok = (loc >= 0) & (loc < n_half)
                meta[1, pl.ds(g * _L, _L)] = jnp.where(
                    ok, loc, n_half + (r16 & 7))

            def scale(j16, c2):
                v16 = plsc.bitcast(meta[2, pl.ds(j16 * _L, _L)], jnp.float32)
                for jj in range(_L):
                    j = j16 * _L + jj
                    bj = lax.gather(
                        v16, jnp.full((_L, 1), jj, jnp.int32),
                        _GATHER_DN, slice_sizes=(1,),
                        mode=lax.GatherScatterMode.PROMISE_IN_BOUNDS)
                    for g in range(_EMB // _L):
                        row[j, pl.ds(g * _L, _L)] = (
                            row[j, pl.ds(g * _L, _L)] * bj
                        )
                return c2

            for j16 in range(_K // _L):
                scale(j16, 0)
            pltpu.sync_copy(row, acc.at[meta.at[1]], add=True)

        def meta_copy(b, s):
            return pltpu.make_async_copy(
                meta_hbm.at[sub, b], metas[s], isems[s])

        def gather_copy(s):
            return pltpu.make_async_copy(
                x_hbm.at[metas[s].at[0]], rows[s], gsems[s])

        pltpu.sync_copy(meta_hbm.at[sub, 0], meta0)
        gather_copy(0).start()
        meta_copy(1, 1).start()

        def pair(i, c):
            b0 = 2 * i
            meta_copy(b0 + 1, 1).wait()
            gather_copy(1).start()
            gather_copy(0).wait()
            process(0)

            @pl.when(i < npair - 1)
            def _():
                meta_copy(b0 + 2, 0).start()

            @pl.when(i < npair - 1)
            def _():
                meta_copy(b0 + 2, 0).wait()
                gather_copy(0).start()

            gather_copy(1).wait()
            process(1)

            @pl.when(i < npair - 1)
            def _():
                meta_copy(b0 + 3, 1).start()

            return c

        lax.fori_loop(0, npair, pair, 0)
        plsc.subcore_barrier()

        o0 = sub * orows_per_tile
        pltpu.sync_copy(acc.at[pl.ds(o0, orows_per_tile)],
                        y_hbm.at[pl.ds(base + o0, orows_per_tile)])
        if otail:
            @pl.when(sub == _NS - 1)
            def _():
                pltpu.sync_copy(
                    acc.at[pl.ds(_NS * orows_per_tile, otail)],
                    y_hbm.at[pl.ds(base + _NS * orows_per_tile, otail)],
                )

    return layer


def _final_body(x0u, x1u, x2u, x3u, x0i, x1i, x2i, x3i, mr, w, b,
                uf, itf, mu, pa, mk):
    ufv = (x0u[...] + x1u[...] + x2u[...] + x3u[...]) * 0.25
    itv = (x0i[...] + x1i[...] + x2i[...] + x3i[...]) * 0.25
    m = (mr[...] > 0.2).astype(jnp.float32)
    muv = ufv * m
    pav = jnp.dot(muv, w[...], preferred_element_type=jnp.float32) + b[...]
    uf[...] = ufv
    itf[...] = itv
    mu[...] = muv
    pa[...] = jnp.maximum(pav, 0.0)
    mk[...] = m


def kernel(user_emb, item_emb, edge_index, adj_values, attr_W, attr_b,
           mask_rand):
    nu, emb = user_emb.shape
    ni = item_emb.shape[0]
    nt = nu + ni
    e = edge_index.shape[1]

    x0 = jnp.concatenate([user_emb, item_emb], axis=0)

    nbl = -(-e // (_NS * _K))
    nb = -(-nbl // 2) * 2
    e_pad = _NS * nb * _K
    pe = e_pad - e
    src_p = jnp.concatenate([edge_index[1], jnp.zeros((pe,), jnp.int32)])
    dst_p = jnp.concatenate([edge_index[0], jnp.full((pe,), -1, jnp.int32)])
    val_p = jnp.concatenate([
        lax.bitcast_convert_type(adj_values, jnp.int32),
        jnp.zeros((pe,), jnp.int32)])
    meta = jnp.stack([src_p.reshape(_NS, nb, _K),
                      dst_p.reshape(_NS, nb, _K),
                      val_p.reshape(_NS, nb, _K)], axis=2)

    layer = _make_layer(nt, nu, nb)
    xs = [x0]
    x = x0
    for _ in range(3):
        x = layer(x, meta)
        xs.append(x)

    blk = 1000
    ngrid = nu // blk
    ublk = pl.BlockSpec((blk, emb), lambda i: (i, 0))
    iblk = pl.BlockSpec((blk, emb), lambda i: (i + ngrid, 0))
    full = pl.BlockSpec((emb, emb), lambda i: (0, 0))
    brow = pl.BlockSpec((1, emb), lambda i: (0, 0))
    o = jax.ShapeDtypeStruct((nu, emb), jnp.float32)

    uf, itf, mu, pa, mk = pl.pallas_call(
        _final_body,
        grid=(ngrid,),
        in_specs=[ublk, ublk, ublk, ublk, iblk, iblk, iblk, iblk,
                  ublk, full, brow],
        out_specs=[ublk, ublk, ublk, ublk, ublk],
        out_shape=[o, o, o, o, o],
    )(xs[0], xs[1], xs[2], xs[3], xs[0], xs[1], xs[2], xs[3],
      mask_rand, attr_W, attr_b.reshape(1, emb))

    return (uf, itf, mu, pa, mk)


# speedup vs baseline: 1.2292x; 1.0004x over previous
import functools

import jax
import jax.numpy as jnp
from jax import lax
from jax.experimental import pallas as pl
from jax.experimental.pallas import tpu as pltpu
from jax.experimental.pallas import tpu_sc as plsc

_NC = 2
_NS = 16
_L = 16
_EMB = 64
_K = 128

_GATHER_DN = lax.GatherDimensionNumbers(
    offset_dims=(), collapsed_slice_dims=(0,), start_index_map=(0,))


@functools.cache
def _make_layer(n_total: int, n_half: int, nb: int):
    zrows_per_tile = (n_half // _NS + 15) // 8 * 8
    r_pad = zrows_per_tile * _NS
    orows_per_tile = n_half // _NS // 8 * 8
    otail = n_half - orows_per_tile * _NS
    npair = nb // 2
    mesh = plsc.VectorSubcoreMesh(
        core_axis_name="c", subcore_axis_name="s", num_cores=_NC
    )

    @functools.partial(
        pl.kernel,
        out_type=jax.ShapeDtypeStruct((n_total, _EMB), jnp.float32),
        mesh=mesh,
        compiler_params=pltpu.CompilerParams(
            use_tc_tiling_on_sc=False, needs_layout_passes=False),
        scratch_types=[
            pltpu.VMEM((3, _K), jnp.int32),
            pltpu.VMEM((3, _K), jnp.int32),
            pltpu.VMEM((_K, _EMB), jnp.float32),
            pltpu.VMEM((_K, _EMB), jnp.float32),
            pltpu.VMEM_SHARED((r_pad, _EMB), jnp.float32),
            pltpu.SemaphoreType.DMA,
            pltpu.SemaphoreType.DMA,
            pltpu.SemaphoreType.DMA,
            pltpu.SemaphoreType.DMA,
        ],
    )
    def layer(x_hbm, meta_hbm, y_hbm,
              meta0, meta1, row0, row1, acc,
              isem0, isem1, gsem0, gsem1):
        core = lax.axis_index("c")
        sub = lax.axis_index("s")
        base = core * n_half
        metas = (meta0, meta1)
        rows = (row0, row1)
        isems = (isem0, isem1)
        gsems = (gsem0, gsem1)

        zero_v = row0
        zv = jnp.zeros((_L,), jnp.float32)

        def zfill(i, c):
            for g in range(_EMB // _L):
                zero_v[i, pl.ds(g * _L, _L)] = zv
            return c

        lax.fori_loop(0, 128, zfill, 0)
        z0 = sub * zrows_per_tile
        nfull = zrows_per_tile // 128
        zrem = zrows_per_tile - nfull * 128

        def zcopy(i, c):
            pltpu.sync_copy(zero_v, acc.at[pl.ds(z0 + i * 128, 128)])
            return c

        lax.fori_loop(0, nfull, zcopy, 0)
        if zrem:
            pltpu.sync_copy(zero_v.at[pl.ds(0, zrem)],
                            acc.at[pl.ds(z0 + nfull * 128, zrem)])
        plsc.subcore_barrier()

        def process(s):
            meta, row = metas[s], rows[s]
            for g in range(_K // _L):
                r16 = meta[1, pl.ds(g * _L, _L)]
                loc = r16 - base
                ok = (loc >= 0) & (loc < n_half)
                meta[1, pl.ds(g * _L, _L)] = jnp.where(
                    ok, loc, n_half + (r16 & 127))

            def scale(j16, c2):
                v16 = plsc.bitcast(meta[2, pl.ds(j16 * _L, _L)], jnp.float32)
                for jj in range(_L):
                    j = j16 * _L + jj
                    bj = lax.gather(
                        v16, jnp.full((_L, 1), jj, jnp.int32),
                        _GATHER_DN, slice_sizes=(1,),
                        mode=lax.GatherScatterMode.PROMISE_IN_BOUNDS)
                    for g in range(_EMB // _L):
                        row[j, pl.ds(g * _L, _L)] = (
                            row[j, pl.ds(g * _L, _L)] * bj
                        )
                return c2

            for j16 in range(_K // _L):
                scale(j16, 0)
            pltpu.sync_copy(row, acc.at[meta.at[1]], add=True)

        def meta_copy(b, s):
            return pltpu.make_async_copy(
                meta_hbm.at[sub, b], metas[s], isems[s])

        def gather_copy(s):
            return pltpu.make_async_copy(
                x_hbm.at[metas[s].at[0]], rows[s], gsems[s])

        pltpu.sync_copy(meta_hbm.at[sub, 0], meta0)
        gather_copy(0).start()
        meta_copy(1, 1).start()

        def pair(i, c):
            b0 = 2 * i
            meta_copy(b0 + 1, 1).wait()
            gather_copy(1).start()
            gather_copy(0).wait()
            process(0)

            @pl.when(i < npair - 1)
            def _():
                meta_copy(b0 + 2, 0).start()

            @pl.when(i < npair - 1)
            def _():
                meta_copy(b0 + 2, 0).wait()
                gather_copy(0).start()

            gather_copy(1).wait()
            process(1)

            @pl.when(i < npair - 1)
            def _():
                meta_copy(b0 + 3, 1).start()

            return c

        lax.fori_loop(0, npair, pair, 0)
        plsc.subcore_barrier()

        o0 = sub * orows_per_tile
        pltpu.sync_copy(acc.at[pl.ds(o0, orows_per_tile)],
                        y_hbm.at[pl.ds(base + o0, orows_per_tile)])
        if otail:
            @pl.when(sub == _NS - 1)
            def _():
                pltpu.sync_copy(
                    acc.at[pl.ds(_NS * orows_per_tile, otail)],
                    y_hbm.at[pl.ds(base + _NS * orows_per_tile, otail)],
                )

    return layer


def _final_body(x0u, x1u, x2u, x3u, x0i, x1i, x2i, x3i, mr, w, b,
                uf, itf, mu, pa, mk):
    ufv = (x0u[...] + x1u[...] + x2u[...] + x3u[...]) * 0.25
    itv = (x0i[...] + x1i[...] + x2i[...] + x3i[...]) * 0.25
    m = (mr[...] > 0.2).astype(jnp.float32)
    muv = ufv * m
    pav = jnp.dot(muv, w[...], preferred_element_type=jnp.float32) + b[...]
    uf[...] = ufv
    itf[...] = itv
    mu[...] = muv
    pa[...] = jnp.maximum(pav, 0.0)
    mk[...] = m


def kernel(user_emb, item_emb, edge_index, adj_values, attr_W, attr_b,
           mask_rand):
    nu, emb = user_emb.shape
    ni = item_emb.shape[0]
    nt = nu + ni
    e = edge_index.shape[1]

    x0 = jnp.concatenate([user_emb, item_emb], axis=0)

    nbl = -(-e // (_NS * _K))
    nb = -(-nbl // 2) * 2
    e_pad = _NS * nb * _K
    pe = e_pad - e
    src_p = jnp.concatenate([edge_index[1], jnp.zeros((pe,), jnp.int32)])
    dst_p = jnp.concatenate([edge_index[0], jnp.full((pe,), -1, jnp.int32)])
    val_p = jnp.concatenate([
        lax.bitcast_convert_type(adj_values, jnp.int32),
        jnp.zeros((pe,), jnp.int32)])
    meta = jnp.stack([src_p.reshape(_NS, nb, _K),
                      dst_p.reshape(_NS, nb, _K),
                      val_p.reshape(_NS, nb, _K)], axis=2)

    layer = _make_layer(nt, nu, nb)
    xs = [x0]
    x = x0
    for _ in range(3):
        x = layer(x, meta)
        xs.append(x)

    blk = 1000
    ngrid = nu // blk
    ublk = pl.BlockSpec((blk, emb), lambda i: (i, 0))
    iblk = pl.BlockSpec((blk, emb), lambda i: (i + ngrid, 0))
    full = pl.BlockSpec((emb, emb), lambda i: (0, 0))
    brow = pl.BlockSpec((1, emb), lambda i: (0, 0))
    o = jax.ShapeDtypeStruct((nu, emb), jnp.float32)

    uf, itf, mu, pa, mk = pl.pallas_call(
        _final_body,
        grid=(ngrid,),
        in_specs=[ublk, ublk, ublk, ublk, iblk, iblk, iblk, iblk,
                  ublk, full, brow],
        out_specs=[ublk, ublk, ublk, ublk, ublk],
        out_shape=[o, o, o, o, o],
    )(xs[0], xs[1], xs[2], xs[3], xs[0], xs[1], xs[2], xs[3],
      mask_rand, attr_W, attr_b.reshape(1, emb))

    return (uf, itf, mu, pa, mk)
